# baseline (speedup 1.0000x reference)
import jax
import jax.numpy as jnp
from jax import lax
from jax.experimental import pallas as pl
from jax.experimental.pallas import tpu as pltpu
from jax.experimental.pallas import tpu_sc as plsc

D_MODEL = 768
SEQ_LEN = 4096
BATCH = 4
NUM_CORES = 2
NUM_SUBCORES = 16
NUM_WORKERS = NUM_CORES * NUM_SUBCORES
L_PER_WORKER = SEQ_LEN // NUM_WORKERS
CHUNK = 8
N_STEPS = L_PER_WORKER // CHUNK
ROWS_PER_STEP = BATCH * CHUNK
NBUF = 4
LANES = 16


def _emb_body(x_hbm, pe_hbm, table_hbm, out_hbm, idx_v, pe_v, rows_v,
              sem_g, sem_st, sem_pe, sem_idx):
    cid = lax.axis_index("c")
    sid = lax.axis_index("s")
    wid = sid * NUM_CORES + cid
    l_base = wid * L_PER_WORKER

    idx_copies = [
        pltpu.async_copy(
            x_hbm.at[pl.ds(b * SEQ_LEN + l_base, L_PER_WORKER)],
            idx_v.at[b], sem_idx)
        for b in range(BATCH)
    ]

    def start_step(s):
        slot = lax.rem(s, NBUF)
        pltpu.async_copy(
            pe_hbm.at[pl.ds(l_base + s * CHUNK, CHUNK)],
            pe_v.at[pl.ds(slot * CHUNK, CHUNK)], sem_pe)
        for b in range(BATCH):
            pltpu.async_copy(
                table_hbm.at[idx_v.at[b, pl.ds(s * CHUNK, CHUNK)]],
                rows_v.at[pl.ds(slot * ROWS_PER_STEP + b * CHUNK, CHUNK)],
                sem_g)

    def wait_pe():
        pltpu.make_async_copy(
            pe_hbm.at[pl.ds(0, CHUNK)], pe_v.at[pl.ds(0, CHUNK)],
            sem_pe).wait()

    def wait_gathers():
        pltpu.make_async_copy(
            table_hbm.at[pl.ds(0, ROWS_PER_STEP)],
            rows_v.at[pl.ds(0, ROWS_PER_STEP)], sem_g).wait()

    def wait_stores():
        pltpu.make_async_copy(
            rows_v.at[pl.ds(0, ROWS_PER_STEP)],
            out_hbm.at[pl.ds(0, ROWS_PER_STEP)], sem_st).wait()

    for c in idx_copies:
        c.wait()

    start_step(0)
    start_step(1)
    start_step(2)

    @pl.loop(0, N_STEPS)
    def _step(s):
        slot = lax.rem(s, NBUF)
        row0 = slot * ROWS_PER_STEP
        pe0 = slot * CHUNK

        wait_pe()
        wait_gathers()

        @pl.when(s >= 1)
        def _():
            wait_stores()

        @pl.when(s + 3 < N_STEPS)
        def _():
            start_step(s + 3)

        @pl.loop(0, CHUNK)
        def _pos(i):
            for j0 in range(0, D_MODEL, 4 * LANES):
                regs = [
                    pe_v[pe0 + i, pl.ds(j0 + k * LANES, LANES)]
                    for k in range(4)
                ]
                for k in range(4):
                    for b in range(BATCH):
                        plsc.addupdate(
                            rows_v.at[row0 + b * CHUNK + i,
                                      pl.ds(j0 + k * LANES, LANES)],
                            regs[k])

        for b in range(BATCH):
            pltpu.async_copy(
                rows_v.at[pl.ds(row0 + b * CHUNK, CHUNK)],
                out_hbm.at[pl.ds(b * SEQ_LEN + l_base + s * CHUNK, CHUNK)],
                sem_st)

    wait_stores()


@jax.jit
def _emb(x_flat, pe, table):
    mesh = plsc.VectorSubcoreMesh(core_axis_name="c", subcore_axis_name="s")
    k = pl.kernel(
        _emb_body,
        out_type=jax.ShapeDtypeStruct((BATCH * SEQ_LEN, D_MODEL), jnp.float32),
        mesh=mesh,
        scratch_types=[
            pltpu.VMEM((BATCH, L_PER_WORKER), jnp.int32),
            pltpu.VMEM((NBUF * CHUNK, D_MODEL), jnp.float32),
            pltpu.VMEM((NBUF * ROWS_PER_STEP, D_MODEL), jnp.float32),
            pltpu.SemaphoreType.DMA,
            pltpu.SemaphoreType.DMA,
            pltpu.SemaphoreType.DMA,
            pltpu.SemaphoreType.DMA,
        ],
    )
    return k(x_flat, pe, table)


def kernel(x, table, pe):
    x_flat = x.reshape(-1).astype(jnp.int32)
    out = _emb(x_flat, pe, table)
    return out.reshape(x.shape[0], x.shape[1], D_MODEL)


# speedup vs baseline: 1.0219x; 1.0219x over previous
import jax
import jax.numpy as jnp
from jax import lax
from jax.experimental import pallas as pl
from jax.experimental.pallas import tpu as pltpu
from jax.experimental.pallas import tpu_sc as plsc

D_MODEL = 768
SEQ_LEN = 4096
BATCH = 4
NUM_CORES = 2
NUM_SUBCORES = 16
NUM_WORKERS = NUM_CORES * NUM_SUBCORES
L_PER_WORKER = SEQ_LEN // NUM_WORKERS
CHUNK = 8
N_STEPS = L_PER_WORKER // CHUNK
ROWS_PER_STEP = BATCH * CHUNK
NBUF = 4
LANES = 16


def _emb_body(x_hbm, pe_hbm, table_hbm, out_hbm, idx_v, pe_v, rows_v,
              sem_g, sem_st, sem_pe, sem_idx):
    cid = lax.axis_index("c")
    sid = lax.axis_index("s")
    wid = sid * NUM_CORES + cid
    l_base = wid * L_PER_WORKER

    idx_copies = [
        pltpu.async_copy(
            x_hbm.at[pl.ds(b * SEQ_LEN + l_base, L_PER_WORKER)],
            idx_v.at[b], sem_idx)
        for b in range(BATCH)
    ]

    def start_step(s):
        slot = lax.rem(s, NBUF)
        pltpu.async_copy(
            pe_hbm.at[pl.ds(l_base + s * CHUNK, CHUNK)],
            pe_v.at[pl.ds(slot * CHUNK, CHUNK)], sem_pe)
        for b in range(BATCH):
            pltpu.async_copy(
                table_hbm.at[idx_v.at[b, pl.ds(s * CHUNK, CHUNK)]],
                rows_v.at[pl.ds(slot * ROWS_PER_STEP + b * CHUNK, CHUNK)],
                sem_g)

    def wait_pe():
        pltpu.make_async_copy(
            pe_hbm.at[pl.ds(0, CHUNK)], pe_v.at[pl.ds(0, CHUNK)],
            sem_pe).wait()

    def wait_gathers():
        pltpu.make_async_copy(
            table_hbm.at[pl.ds(0, ROWS_PER_STEP)],
            rows_v.at[pl.ds(0, ROWS_PER_STEP)], sem_g).wait()

    def wait_stores():
        pltpu.make_async_copy(
            rows_v.at[pl.ds(0, ROWS_PER_STEP)],
            out_hbm.at[pl.ds(0, ROWS_PER_STEP)], sem_st).wait()

    for c in idx_copies:
        c.wait()

    start_step(0)
    start_step(1)

    @pl.loop(0, N_STEPS)
    def _step(s):
        slot = lax.rem(s, NBUF)
        row0 = slot * ROWS_PER_STEP
        pe0 = slot * CHUNK

        wait_pe()
        wait_gathers()

        @pl.when(s >= 2)
        def _():
            wait_stores()

        @pl.when(s + 2 < N_STEPS)
        def _():
            start_step(s + 2)

        @pl.loop(0, CHUNK)
        def _pos(i):
            for j0 in range(0, D_MODEL, 4 * LANES):
                regs = [
                    pe_v[pe0 + i, pl.ds(j0 + k * LANES, LANES)]
                    for k in range(4)
                ]
                for k in range(4):
                    for b in range(BATCH):
                        plsc.addupdate(
                            rows_v.at[row0 + b * CHUNK + i,
                                      pl.ds(j0 + k * LANES, LANES)],
                            regs[k])

        for b in range(BATCH):
            pltpu.async_copy(
                rows_v.at[pl.ds(row0 + b * CHUNK, CHUNK)],
                out_hbm.at[pl.ds(b * SEQ_LEN + l_base + s * CHUNK, CHUNK)],
                sem_st)

    wait_stores()
    wait_stores()


@jax.jit
def _emb(x_flat, pe, table):
    mesh = plsc.VectorSubcoreMesh(core_axis_name="c", subcore_axis_name="s")
    k = pl.kernel(
        _emb_body,
        out_type=jax.ShapeDtypeStruct((BATCH * SEQ_LEN, D_MODEL), jnp.float32),
        mesh=mesh,
        scratch_types=[
            pltpu.VMEM((BATCH, L_PER_WORKER), jnp.int32),
            pltpu.VMEM((NBUF * CHUNK, D_MODEL), jnp.float32),
            pltpu.VMEM((NBUF * ROWS_PER_STEP, D_MODEL), jnp.float32),
            pltpu.SemaphoreType.DMA,
            pltpu.SemaphoreType.DMA,
            pltpu.SemaphoreType.DMA,
            pltpu.SemaphoreType.DMA,
        ],
    )
    return k(x_flat, pe, table)


def kernel(x, table, pe):
    x_flat = x.reshape(-1).astype(jnp.int32)
    out = _emb(x_flat, pe, table)
    return out.reshape(x.shape[0], x.shape[1], D_MODEL)


# speedup vs baseline: 1.0268x; 1.0048x over previous
import jax
import jax.numpy as jnp
from jax import lax
from jax.experimental import pallas as pl
from jax.experimental.pallas import tpu as pltpu
from jax.experimental.pallas import tpu_sc as plsc

D_MODEL = 768
SEQ_LEN = 4096
BATCH = 4
NUM_CORES = 2
NUM_SUBCORES = 16
NUM_WORKERS = NUM_CORES * NUM_SUBCORES
L_PER_WORKER = SEQ_LEN // NUM_WORKERS
CHUNK = 8
N_STEPS = L_PER_WORKER // CHUNK
ROWS_PER_STEP = BATCH * CHUNK
NBUF = 4
LANES = 16


def _emb_body(x_hbm, pe_hbm, table_hbm, out_hbm, idx_v, pe_v, rows_v,
              sem_g, sem_st, sem_pe, sem_idx):
    cid = lax.axis_index("c")
    sid = lax.axis_index("s")
    wid = sid * NUM_CORES + cid
    l_base = wid * L_PER_WORKER

    idx_copies = [
        pltpu.async_copy(
            x_hbm.at[pl.ds(b * SEQ_LEN + l_base, L_PER_WORKER)],
            idx_v.at[b], sem_idx)
        for b in range(BATCH)
    ]

    def start_step(s):
        slot = lax.rem(s, NBUF)
        pltpu.async_copy(
            pe_hbm.at[pl.ds(l_base + s * CHUNK, CHUNK)],
            pe_v.at[pl.ds(slot * CHUNK, CHUNK)], sem_pe)
        for b in range(BATCH):
            pltpu.async_copy(
                table_hbm.at[idx_v.at[b, pl.ds(s * CHUNK, CHUNK)]],
                rows_v.at[pl.ds(slot * ROWS_PER_STEP + b * CHUNK, CHUNK)],
                sem_g)

    def wait_pe():
        pltpu.make_async_copy(
            pe_hbm.at[pl.ds(0, CHUNK)], pe_v.at[pl.ds(0, CHUNK)],
            sem_pe).wait()

    def wait_gathers():
        pltpu.make_async_copy(
            table_hbm.at[pl.ds(0, ROWS_PER_STEP)],
            rows_v.at[pl.ds(0, ROWS_PER_STEP)], sem_g).wait()

    def wait_stores():
        pltpu.make_async_copy(
            rows_v.at[pl.ds(0, ROWS_PER_STEP)],
            out_hbm.at[pl.ds(0, ROWS_PER_STEP)], sem_st).wait()

    for c in idx_copies:
        c.wait()

    start_step(0)
    start_step(1)

    @pl.loop(0, N_STEPS)
    def _step(s):
        slot = lax.rem(s, NBUF)
        row0 = slot * ROWS_PER_STEP
        pe0 = slot * CHUNK

        wait_pe()
        wait_gathers()

        @pl.when(s >= 2)
        def _():
            wait_stores()

        @pl.when(s + 2 < N_STEPS)
        def _():
            start_step(s + 2)

        @plsc.parallel_loop(0, CHUNK, unroll=2)
        def _pos(i):
            for j0 in range(0, D_MODEL, 4 * LANES):
                regs = [
                    pe_v[pe0 + i, pl.ds(j0 + k * LANES, LANES)]
                    for k in range(4)
                ]
                for k in range(4):
                    for b in range(BATCH):
                        plsc.addupdate(
                            rows_v.at[row0 + b * CHUNK + i,
                                      pl.ds(j0 + k * LANES, LANES)],
                            regs[k])

        for b in range(BATCH):
            pltpu.async_copy(
                rows_v.at[pl.ds(row0 + b * CHUNK, CHUNK)],
                out_hbm.at[pl.ds(b * SEQ_LEN + l_base + s * CHUNK, CHUNK)],
                sem_st)

    wait_stores()
    wait_stores()


@jax.jit
def _emb(x_flat, pe, table):
    mesh = plsc.VectorSubcoreMesh(core_axis_name="c", subcore_axis_name="s")
    k = pl.kernel(
        _emb_body,
        out_type=jax.ShapeDtypeStruct((BATCH * SEQ_LEN, D_MODEL), jnp.float32),
        mesh=mesh,
        scratch_types=[
            pltpu.VMEM((BATCH, L_PER_WORKER), jnp.int32),
            pltpu.VMEM((NBUF * CHUNK, D_MODEL), jnp.float32),
            pltpu.VMEM((NBUF * ROWS_PER_STEP, D_MODEL), jnp.float32),
            pltpu.SemaphoreType.DMA,
            pltpu.SemaphoreType.DMA,
            pltpu.SemaphoreType.DMA,
            pltpu.SemaphoreType.DMA,
        ],
    )
    return k(x_flat, pe, table)


def kernel(x, table, pe):
    x_flat = x.reshape(-1).astype(jnp.int32)
    out = _emb(x_flat, pe, table)
    return out.reshape(x.shape[0], x.shape[1], D_MODEL)
